# baseline (speedup 1.0000x reference)
import functools

import jax
import jax.numpy as jnp
from jax import lax
from jax.experimental import pallas as pl
from jax.experimental.pallas import tpu as pltpu
from jax.experimental.pallas import tpu_sc as plsc

N = 10000
C = 128
H = 512
NPAD = 10112
NS = 16
CHUNK = 128
KPT = 162
RB = 3
SR = 6
TOT_CHUNKS = NS * KPT
EPAD = TOT_CHUNKS * CHUNK
ROWS_PER_TILE = NPAD // NS
ACC_ROWS = 10008
LAST_LEN = ACC_ROWS - 15 * ROWS_PER_TILE
BN = 632


def _sc_agg(x_pad, src_idx, dst_idx):
    mesh = plsc.VectorSubcoreMesh(core_axis_name="c", subcore_axis_name="s")

    @functools.partial(
        pl.kernel,
        out_type=jax.ShapeDtypeStruct((NPAD, C), jnp.float32),
        mesh=mesh,
        scratch_types=[
            pltpu.VMEM((SR, CHUNK), jnp.int32),
            pltpu.VMEM((2, CHUNK), jnp.int32),
            pltpu.VMEM((RB, CHUNK, C), jnp.float32),
            pltpu.VMEM_SHARED((ACC_ROWS, C), jnp.float32),
            dict(g=(pltpu.SemaphoreType.DMA,) * RB,
                 d=(pltpu.SemaphoreType.DMA,) * 2,
                 s=(pltpu.SemaphoreType.DMA,) * SR),
        ],
    )
    def sc_agg(x_hbm, src_hbm, dst_hbm, out_hbm,
               src_v, dst_v, rows_v, acc_sh, sems):
        cid = lax.axis_index("c")
        sid = lax.axis_index("s")
        gsems, dsems, ssems = sems["g"], sems["d"], sems["s"]
        row0 = sid * ROWS_PER_TILE

        @pl.when(cid == 0)
        def _():
            with jax.named_scope("agg_stage"):
                @pl.when(sid < NS - 1)
                def _():
                    pltpu.sync_copy(x_hbm.at[pl.ds(row0, ROWS_PER_TILE)],
                                    acc_sh.at[pl.ds(row0, ROWS_PER_TILE)])

                @pl.when(sid == NS - 1)
                def _():
                    pltpu.sync_copy(x_hbm.at[pl.ds(row0, LAST_LEN)],
                                    acc_sh.at[pl.ds(row0, LAST_LEN)])

        plsc.subcore_barrier()

        base = sid * KPT

        def scp(j, sb):
            return pltpu.make_async_copy(
                src_hbm.at[base + j], src_v.at[sb], ssems[sb])

        def dcp(j, db):
            return pltpu.make_async_copy(
                dst_hbm.at[base + j], dst_v.at[db], dsems[db])

        def gather(rb, sb):
            return pltpu.make_async_copy(
                x_hbm.at[src_v.at[sb]], rows_v.at[rb], gsems[rb])

        @pl.when(cid == 0)
        def _():
            with jax.named_scope("agg_loop"):
                for j in range(SR):
                    scp(j, j).start()
                dcp(0, 0).start()
                dcp(1, 1).start()
                for j in range(RB):
                    scp(j, j).wait()
                    gather(j, j).start()

                def body(j6, carry):
                    for b6 in range(6):
                        ch = j6 * 6 + b6
                        rb = b6 % RB
                        sb = b6 % SR
                        db = b6 % 2
                        gather(rb, sb).wait()
                        dcp(ch, db).wait()
                        pltpu.sync_copy(rows_v.at[rb],
                                        acc_sh.at[dst_v.at[db]], add=True)

                        @pl.when(ch + SR < KPT)
                        def _():
                            scp(ch + SR, sb).start()

                        @pl.when(ch + RB < KPT)
                        def _():
                            scp(ch + RB, (b6 + RB) % SR).wait()
                            gather(rb, (b6 + RB) % SR).start()

                        @pl.when(ch + 2 < KPT)
                        def _():
                            dcp(ch + 2, db).start()
                    return carry

                lax.fori_loop(0, KPT // 6, body, 0)

        plsc.subcore_barrier()

        @pl.when(cid == 0)
        def _():
            with jax.named_scope("agg_publish"):
                @pl.when(sid < NS - 1)
                def _():
                    pltpu.sync_copy(acc_sh.at[pl.ds(row0, ROWS_PER_TILE)],
                                    out_hbm.at[pl.ds(row0, ROWS_PER_TILE)])

                @pl.when(sid == NS - 1)
                def _():
                    pltpu.sync_copy(acc_sh.at[pl.ds(row0, LAST_LEN)],
                                    out_hbm.at[pl.ds(row0, LAST_LEN)])

    return sc_agg(x_pad, src_idx, dst_idx)


def _mlp(agg, Wa, ba, Wb, bb):
    def body(a_ref, wa_ref, ba_ref, wb_ref, bb_ref, o_ref):
        h = a_ref[...]
        z = jnp.dot(h, wa_ref[...], preferred_element_type=jnp.float32)
        z = jnp.maximum(z + ba_ref[...], 0.0)
        y = jnp.dot(z, wb_ref[...], preferred_element_type=jnp.float32)
        y = y + bb_ref[...]
        rows = pl.program_id(0) * BN + lax.broadcasted_iota(
            jnp.int32, (BN, 1), 0)
        o_ref[...] = jnp.where(rows < N, y, 0.0)

    return pl.pallas_call(
        body,
        grid=(NPAD // BN,),
        in_specs=[
            pl.BlockSpec((BN, C), lambda i: (i, 0)),
            pl.BlockSpec((C, H), lambda i: (0, 0)),
            pl.BlockSpec((1, H), lambda i: (0, 0)),
            pl.BlockSpec((H, C), lambda i: (0, 0)),
            pl.BlockSpec((1, C), lambda i: (0, 0)),
        ],
        out_specs=pl.BlockSpec((BN, C), lambda i: (i, 0)),
        out_shape=jax.ShapeDtypeStruct((NPAD, C), jnp.float32),
    )(agg, Wa, ba.reshape(1, H), Wb, bb.reshape(1, C))


def kernel(graph_sig, edge_index, W1, b1, W2, b2, W3, b3, W4, b4):
    x0 = graph_sig[0].astype(jnp.float32)
    x_pad = jnp.zeros((NPAD, C), jnp.float32).at[:N].set(x0)

    E = edge_index.shape[1]
    ei = edge_index.astype(jnp.int32)
    fill = jnp.full((EPAD - E,), N, jnp.int32)
    src_idx = jnp.concatenate([ei[0], fill]).reshape(TOT_CHUNKS, CHUNK)
    dst_idx = jnp.concatenate([ei[1], fill]).reshape(TOT_CHUNKS, CHUNK)

    y1 = _mlp(_sc_agg(x_pad, src_idx, dst_idx), W1, b1, W2, b2)
    y2 = _mlp(_sc_agg(y1, src_idx, dst_idx), W3, b3, W4, b4)
    return y2[:N][None]


# speedup vs baseline: 1.4790x; 1.4790x over previous
import functools

import jax
import jax.numpy as jnp
from jax import lax
from jax.experimental import pallas as pl
from jax.experimental.pallas import tpu as pltpu
from jax.experimental.pallas import tpu_sc as plsc

N = 10000
C = 128
H = 512
NPAD = 10240
NC = 2
NS = 16
NW = NC * NS
CHUNK = 128
K = 80
EPW = K * CHUNK
EPAD = NW * EPW
ROWS_PER_TILE = NPAD // NS


def _sc_partial_agg(x_pad, src_idx, dst_idx, zeros_pad):
    mesh = plsc.VectorSubcoreMesh(core_axis_name="c", subcore_axis_name="s")

    @functools.partial(
        pl.kernel,
        out_type=jax.ShapeDtypeStruct((NC, NPAD, C), jnp.float32),
        mesh=mesh,
        scratch_types=[
            pltpu.VMEM((K, CHUNK), jnp.int32),
            pltpu.VMEM((2, CHUNK), jnp.int32),
            pltpu.VMEM((2, CHUNK, C), jnp.float32),
            pltpu.VMEM_SHARED((NPAD, C), jnp.float32),
            pltpu.SemaphoreType.DMA,
            pltpu.SemaphoreType.DMA,
            pltpu.SemaphoreType.DMA,
            pltpu.SemaphoreType.DMA,
        ],
    )
    def sc_agg(x_hbm, src_hbm, dst_hbm, zero_hbm, out_hbm,
               src_v, dst_v, rows_v, acc_sh, gsem0, gsem1, dsem0, dsem1):
        cid = lax.axis_index("c")
        sid = lax.axis_index("s")
        wid = cid * NS + sid
        gsems = (gsem0, gsem1)
        dsems = (dsem0, dsem1)

        with jax.named_scope("agg_stage"):
            pltpu.sync_copy(src_hbm.at[wid], src_v)
            row0 = sid * ROWS_PER_TILE
            pltpu.sync_copy(zero_hbm.at[pl.ds(row0, ROWS_PER_TILE)],
                            acc_sh.at[pl.ds(row0, ROWS_PER_TILE)])
            plsc.subcore_barrier()

        def gather(j, b):
            return pltpu.make_async_copy(
                x_hbm.at[src_v.at[j]], rows_v.at[b], gsems[b])

        def dcp(j, b):
            return pltpu.make_async_copy(
                dst_hbm.at[wid, j], dst_v.at[b], dsems[b])

        gather(0, 0).start()
        dcp(0, 0).start()
        gather(1, 1).start()
        dcp(1, 1).start()

        def body(j2, carry):
            for b in range(2):
                ch = j2 * 2 + b
                gather(ch, b).wait()
                dcp(ch, b).wait()
                pltpu.sync_copy(rows_v.at[b], acc_sh.at[dst_v.at[b]],
                                add=True)

                @pl.when(ch + 2 < K)
                def _():
                    gather(ch + 2, b).start()
                    dcp(ch + 2, b).start()
            return carry

        with jax.named_scope("agg_loop"):
            lax.fori_loop(0, K // 2, body, 0)
            plsc.subcore_barrier()
        with jax.named_scope("agg_publish"):
            pltpu.sync_copy(acc_sh.at[pl.ds(row0, ROWS_PER_TILE)],
                            out_hbm.at[cid, pl.ds(row0, ROWS_PER_TILE)])

    return sc_agg(x_pad, src_idx, dst_idx, zeros_pad)


def _mlp(x_pad, a0, a1, Wa, ba, Wb, bb):
    BN = 512

    def body(x_ref, a0_ref, a1_ref, wa_ref, ba_ref, wb_ref, bb_ref, o_ref):
        h = x_ref[...] + a0_ref[...] + a1_ref[...]
        z = jnp.dot(h, wa_ref[...], preferred_element_type=jnp.float32)
        z = jnp.maximum(z + ba_ref[...], 0.0)
        y = jnp.dot(z, wb_ref[...], preferred_element_type=jnp.float32)
        y = y + bb_ref[...]
        rows = pl.program_id(0) * BN + lax.broadcasted_iota(
            jnp.int32, (BN, 1), 0)
        o_ref[...] = jnp.where(rows < N, y, 0.0)

    return pl.pallas_call(
        body,
        grid=(NPAD // BN,),
        in_specs=[
            pl.BlockSpec((BN, C), lambda i: (i, 0)),
            pl.BlockSpec((BN, C), lambda i: (i, 0)),
            pl.BlockSpec((BN, C), lambda i: (i, 0)),
            pl.BlockSpec((C, H), lambda i: (0, 0)),
            pl.BlockSpec((1, H), lambda i: (0, 0)),
            pl.BlockSpec((H, C), lambda i: (0, 0)),
            pl.BlockSpec((1, C), lambda i: (0, 0)),
        ],
        out_specs=pl.BlockSpec((BN, C), lambda i: (i, 0)),
        out_shape=jax.ShapeDtypeStruct((NPAD, C), jnp.float32),
    )(x_pad, a0, a1, Wa, ba.reshape(1, H), Wb, bb.reshape(1, C))


def kernel(graph_sig, edge_index, W1, b1, W2, b2, W3, b3, W4, b4):
    x0 = graph_sig[0].astype(jnp.float32)
    x_pad = jnp.zeros((NPAD, C), jnp.float32).at[:N].set(x0)

    E = edge_index.shape[1]
    ei = edge_index.astype(jnp.int32)
    fill = jnp.full((EPAD - E,), N, jnp.int32)
    src_idx = jnp.concatenate([ei[0], fill]).reshape(NW, K, CHUNK)
    dst_idx = jnp.concatenate([ei[1], fill]).reshape(NW, K, CHUNK)
    zeros_pad = jnp.zeros((NPAD, C), jnp.float32)

    agg = _sc_partial_agg(x_pad, src_idx, dst_idx, zeros_pad)
    y1 = _mlp(x_pad, agg[0], agg[1], W1, b1, W2, b2)
    agg2 = _sc_partial_agg(y1, src_idx, dst_idx, zeros_pad)
    y2 = _mlp(y1, agg2[0], agg2[1], W3, b3, W4, b4)
    return y2[:N][None]
